# baseline (speedup 1.0000x reference)
import functools

import jax
import jax.numpy as jnp
from jax import lax
from jax.experimental import pallas as pl
from jax.experimental.pallas import tpu as pltpu
from jax.experimental.pallas import tpu_sc as plsc

N = 10000
E = 160000
D = 256
H0 = 512
H1 = 256
H2 = 512
OUT = 64

NPAD = 10240
NTILE = 16
NCORE = 2
NW = NTILE * NCORE
RPW = NPAD // NW
ACC_ROWS = RPW + 8
TRASH = RPW
CHUNK = 1280
NCHUNK = E // CHUNK
BK = 64
BKLOG = 6
CAP = 2048
NB2 = CAP // BK
NZB = RPW // 64
DEGW = 256

_mesh = plsc.VectorSubcoreMesh(
    core_axis_name="c", subcore_axis_name="s", num_cores=NCORE,
    num_subcores=NTILE)


def _scalar(vec, i):
    return jnp.squeeze(lax.slice(vec, (i,), (i + 1,)))


def _agg_pass(feat_h, edges_h, sum_out, g, ebuf,
              csrc, cdst, rows, acc, zidx, sem, cnt=None):
    iota16 = lax.iota(jnp.int32, 16)
    ftrue = iota16 >= 0
    one16 = jnp.where(iota16 == 0, 1.0, 0.0)
    z16 = jnp.zeros((16,), jnp.float32)
    W = rows.shape[1]

    def zacc(i, _):
        for j in range(W // 16):
            acc[i, pl.ds(j * 16, 16)] = z16
        return 0

    lax.fori_loop(0, ACC_ROWS, zacc, 0)
    if cnt is not None:
        def zcnt(i, _):
            for j in range(8):
                cnt[i, pl.ds(j * 16, 16)] = z16
            return 0

        lax.fori_loop(0, ACC_ROWS // 8, zcnt, 0)

    def batch(br, _):
        pltpu.async_copy(feat_h.at[csrc.at[br]], rows, sem).wait()

        for sub in range(BK // 16):
            dv = cdst[br, pl.ds(sub * 16, 16)]
            for i in range(16):
                d = _scalar(dv, i)
                r = sub * 16 + i
                for j in range(W // 16):
                    plsc.addupdate(acc.at[d, pl.ds(j * 16, 16)],
                                   rows[r, pl.ds(j * 16, 16)])
                if cnt is not None:
                    plsc.addupdate(
                        cnt.at[d >> 3, pl.ds((d & 7) * 16, 16)], one16)
        return 0

    def chunk_one(ch, carry):
        cur, done = carry
        pltpu.sync_copy(edges_h.at[pl.ds(ch * 2 * CHUNK, 2 * CHUNK)], ebuf)

        def step(i, c):
            sv = ebuf[pl.ds(i * 16, 16)]
            dv = ebuf[pl.ds(CHUNK + i * 16, 16)]
            m = (dv & (NW - 1)) == g
            cs = plsc.cumsum(jnp.where(m, 1, 0))
            pos = c + cs - 1
            pw = pos & (CAP - 1)
            plsc.store_scatter(cdst, [pw >> BKLOG, pw & (BK - 1)],
                               dv >> 5, mask=m)
            plsc.store_scatter(csrc, [pw >> BKLOG, pw & (BK - 1)],
                               sv, mask=m)
            return c + _scalar(cs, 15)

        cur = lax.fori_loop(0, CHUNK // 16, step, cur)
        for j in range(BK // 16):
            p = cur + j * 16 + iota16
            pw = p & (CAP - 1)
            plsc.store_scatter(cdst, [pw >> BKLOG, pw & (BK - 1)],
                               jnp.full((16,), TRASH, jnp.int32), mask=ftrue)
            plsc.store_scatter(csrc, [pw >> BKLOG, pw & (BK - 1)],
                               jnp.zeros((16,), jnp.int32), mask=ftrue)
        extra = jnp.where(ch == NCHUNK - 1, BK - 1, 0)
        nb_d = (cur - done + extra) >> BKLOG

        def drain(t, _):
            batch(((done >> BKLOG) + t) & (NB2 - 1), 0)
            return 0

        lax.fori_loop(0, nb_d, drain, 0)
        return (cur, done + (nb_d << BKLOG))

    cur, done = lax.fori_loop(0, NCHUNK, chunk_one,
                              (jnp.int32(0), jnp.int32(0)))

    def obatch(b, _):
        pltpu.sync_copy(acc.at[pl.ds(b * 64, 64)], sum_out.at[zidx.at[b]])
        return 0

    lax.fori_loop(0, NZB, obatch, 0)


def _build_zidx(zidx, g):
    iota16 = lax.iota(jnp.int32, 16)
    ftrue = iota16 >= 0

    def zi(i, _):
        p = i * 16 + iota16
        plsc.store_scatter(zidx, [p >> 6, p & 63], g + NW * p,
                           mask=ftrue)
        return 0

    lax.fori_loop(0, RPW // 16, zi, 0)


def _sc1_body(feat_h, edges_h, sum_out, deg_out,
              ebuf, csrc, cdst, rows, acc, cnt, zidx, sem):
    c = lax.axis_index("c")
    s = lax.axis_index("s")
    g = s * NCORE + c
    _build_zidx(zidx, g)

    _agg_pass(feat_h, edges_h, sum_out, g, ebuf,
              csrc, cdst, rows, acc, zidx, sem, cnt=cnt)

    z16 = jnp.zeros((16,), jnp.float32)

    def dbatch(b, _):
        def crow(i, _):
            l = b * BK + i
            rows[i, pl.ds(0, 16)] = cnt[l >> 3, pl.ds((l & 7) * 16, 16)]
            for j in range(1, DEGW // 16):
                rows[i, pl.ds(j * 16, 16)] = z16
            return 0

        lax.fori_loop(0, 64, crow, 0)
        pltpu.sync_copy(rows.at[pl.ds(0, 64)], deg_out.at[zidx.at[b]])
        return 0

    lax.fori_loop(0, NZB, dbatch, 0)


_sc_agg1 = functools.partial(
    pl.kernel,
    out_type=(jax.ShapeDtypeStruct((NPAD, D), jnp.float32),
              jax.ShapeDtypeStruct((NPAD, DEGW), jnp.float32)),
    mesh=_mesh,
    compiler_params=pltpu.CompilerParams(needs_layout_passes=False),
    scratch_types=[
        pltpu.VMEM((2 * CHUNK,), jnp.int32),
        pltpu.VMEM((NB2, BK), jnp.int32),
        pltpu.VMEM((NB2, BK), jnp.int32),
        pltpu.VMEM((BK, D), jnp.float32),
        pltpu.VMEM((ACC_ROWS, D), jnp.float32),
        pltpu.VMEM((ACC_ROWS // 8, 128), jnp.float32),
        pltpu.VMEM((NZB, 64), jnp.int32),
        pltpu.SemaphoreType.DMA,
    ],
)(_sc1_body)


def _sc2_body(h0_h, h1_h, edges_h, s2a_out, s2b_out,
              ebuf, csrc, cdst, rows, acc, zidx, sem):
    c = lax.axis_index("c")
    s = lax.axis_index("s")
    g = s * NCORE + c
    _build_zidx(zidx, g)
    for feat_h, sout in ((h0_h, s2a_out), (h1_h, s2b_out)):
        _agg_pass(feat_h, edges_h, sout, g, ebuf,
                  csrc, cdst, rows, acc, zidx, sem)


_sc_agg2 = functools.partial(
    pl.kernel,
    out_type=(jax.ShapeDtypeStruct((NPAD, D), jnp.float32),
              jax.ShapeDtypeStruct((NPAD, D), jnp.float32)),
    mesh=_mesh,
    compiler_params=pltpu.CompilerParams(needs_layout_passes=False),
    scratch_types=[
        pltpu.VMEM((2 * CHUNK,), jnp.int32),
        pltpu.VMEM((NB2, BK), jnp.int32),
        pltpu.VMEM((NB2, BK), jnp.int32),
        pltpu.VMEM((BK, D), jnp.float32),
        pltpu.VMEM((ACC_ROWS, D), jnp.float32),
        pltpu.VMEM((NZB, 64), jnp.int32),
        pltpu.SemaphoreType.DMA,
    ],
)(_sc2_body)


BM = 512


def _dot(a, b):
    return lax.dot_general(a, b, (((1,), (0,)), ((), ())),
                           precision=lax.Precision.HIGHEST,
                           preferred_element_type=jnp.float32)


def _sigmoid(x):
    return 1.0 / (1.0 + jnp.exp(-x))


def _tc1_body(x_ref, s_ref, d_ref, w1a_ref, w1b_ref, b1_ref,
              h0_ref, h1_ref):
    inv_deg = 1.0 / jnp.maximum(d_ref[:, 0:1], 1.0)
    mean = s_ref[...] * inv_deg
    acc = _dot(x_ref[...], w1a_ref[...]) + _dot(mean, w1b_ref[...])
    h = _sigmoid(acc + b1_ref[...])
    h0_ref[...] = h[:, :D]
    h1_ref[...] = h[:, D:]


def _tc1(featp, sum1, deg, w1a, w1b, b1r):
    return pl.pallas_call(
        _tc1_body,
        grid=(NPAD // BM,),
        in_specs=[
            pl.BlockSpec((BM, D), lambda i: (i, 0)),
            pl.BlockSpec((BM, D), lambda i: (i, 0)),
            pl.BlockSpec((BM, DEGW), lambda i: (i, 0)),
            pl.BlockSpec((D, H0), lambda i: (0, 0)),
            pl.BlockSpec((D, H0), lambda i: (0, 0)),
            pl.BlockSpec((1, H0), lambda i: (0, 0)),
        ],
        out_specs=[pl.BlockSpec((BM, D), lambda i: (i, 0)),
                   pl.BlockSpec((BM, D), lambda i: (i, 0))],
        out_shape=[jax.ShapeDtypeStruct((NPAD, D), jnp.float32),
                   jax.ShapeDtypeStruct((NPAD, D), jnp.float32)],
    )(featp, sum1, deg, w1a, w1b, b1r)


def _tc2_body(h0_ref, h1_ref, sa_ref, sb_ref, d_ref,
              w2a0_ref, w2a1_ref, w2b0_ref, w2b1_ref, b2_ref,
              wm1_ref, bm1_ref, wm2_ref, bm2_ref, o_ref):
    inv_deg = 1.0 / jnp.maximum(d_ref[:, 0:1], 1.0)
    h2 = _sigmoid(_dot(h0_ref[...], w2a0_ref[...])
                  + _dot(h1_ref[...], w2a1_ref[...])
                  + _dot(sa_ref[...] * inv_deg, w2b0_ref[...])
                  + _dot(sb_ref[...] * inv_deg, w2b1_ref[...])
                  + b2_ref[...])
    z = jnp.maximum(_dot(h2, wm1_ref[...]) + bm1_ref[...], 0.0)
    o_ref[...] = _dot(z, wm2_ref[...]) + bm2_ref[...]


def _tc2(h0, h1, s2a, s2b, deg, w2a0, w2a1, w2b0, w2b1, b2r,
         wm1, bm1r, wm2, bm2r):
    row = lambda i: (i, 0)
    fix = lambda i: (0, 0)
    return pl.pallas_call(
        _tc2_body,
        grid=(NPAD // BM,),
        in_specs=[
            pl.BlockSpec((BM, D), row),
            pl.BlockSpec((BM, D), row),
            pl.BlockSpec((BM, D), row),
            pl.BlockSpec((BM, D), row),
            pl.BlockSpec((BM, DEGW), row),
            pl.BlockSpec((D, H1), fix),
            pl.BlockSpec((D, H1), fix),
            pl.BlockSpec((D, H1), fix),
            pl.BlockSpec((D, H1), fix),
            pl.BlockSpec((1, H1), fix),
            pl.BlockSpec((H1, H2), fix),
            pl.BlockSpec((1, H2), fix),
            pl.BlockSpec((H2, OUT), fix),
            pl.BlockSpec((1, OUT), fix),
        ],
        out_specs=pl.BlockSpec((BM, OUT), row),
        out_shape=jax.ShapeDtypeStruct((NPAD, OUT), jnp.float32),
    )(h0, h1, s2a, s2b, deg, w2a0, w2a1, w2b0, w2b1, b2r,
      wm1, bm1r, wm2, bm2r)


def kernel(features, edge_index, W1, b1, W2, b2, Wm1, bm1, Wm2, bm2):
    src = edge_index[0]
    dst = edge_index[1]
    e2 = jnp.concatenate([src.reshape(NCHUNK, CHUNK),
                          dst.reshape(NCHUNK, CHUNK)], axis=1).reshape(-1)
    featp = jnp.zeros((NPAD, D), jnp.float32).at[:N].set(features)

    sum1, deg = _sc_agg1(features, e2)
    h0, h1 = _tc1(featp, sum1, deg, W1[:D], W1[D:], b1.reshape(1, H0))
    s2a, s2b = _sc_agg2(h0, h1, e2)
    out = _tc2(h0, h1, s2a, s2b, deg,
               W2[0:D], W2[D:2 * D], W2[2 * D:3 * D], W2[3 * D:4 * D],
               b2.reshape(1, H1), Wm1, bm1.reshape(1, H2),
               Wm2, bm2.reshape(1, OUT))
    return out[:N]


# speedup vs baseline: 1.0160x; 1.0160x over previous
import functools

import jax
import jax.numpy as jnp
from jax import lax
from jax.experimental import pallas as pl
from jax.experimental.pallas import tpu as pltpu
from jax.experimental.pallas import tpu_sc as plsc

N = 10000
E = 160000
D = 256
H0 = 512
H1 = 256
H2 = 512
OUT = 64

NPAD = 10240
NTILE = 16
NCORE = 2
NW = NTILE * NCORE
RPW = NPAD // NW
ACC_ROWS = RPW + 8
TRASH = RPW
CHUNK = 1600
NCHUNK = E // CHUNK
BK = 64
BKLOG = 6
CAP = 2048
NB2 = CAP // BK
NZB = RPW // 64
DEGW = 256

_mesh = plsc.VectorSubcoreMesh(
    core_axis_name="c", subcore_axis_name="s", num_cores=NCORE,
    num_subcores=NTILE)


def _scalar(vec, i):
    return jnp.squeeze(lax.slice(vec, (i,), (i + 1,)))


def _agg_pass(feat_h, edges_h, sum_out, g, ebuf,
              csrc, cdst, rows, acc, zidx, sem, cnt=None):
    iota16 = lax.iota(jnp.int32, 16)
    ftrue = iota16 >= 0
    one16 = jnp.where(iota16 == 0, 1.0, 0.0)
    z16 = jnp.zeros((16,), jnp.float32)
    W = rows.shape[1]

    def zacc(i, _):
        for j in range(W // 16):
            acc[i, pl.ds(j * 16, 16)] = z16
        return 0

    lax.fori_loop(0, ACC_ROWS, zacc, 0)
    if cnt is not None:
        def zcnt(i, _):
            for j in range(8):
                cnt[i, pl.ds(j * 16, 16)] = z16
            return 0

        lax.fori_loop(0, ACC_ROWS // 8, zcnt, 0)

    def batch(br, _):
        pltpu.async_copy(feat_h.at[csrc.at[br]], rows, sem).wait()

        for sub in range(BK // 16):
            dv = cdst[br, pl.ds(sub * 16, 16)]
            for i in range(16):
                d = _scalar(dv, i)
                r = sub * 16 + i
                for j in range(W // 16):
                    plsc.addupdate(acc.at[d, pl.ds(j * 16, 16)],
                                   rows[r, pl.ds(j * 16, 16)])
                if cnt is not None:
                    plsc.addupdate(
                        cnt.at[d >> 3, pl.ds((d & 7) * 16, 16)], one16)
        return 0

    def chunk_one(ch, carry):
        cur, done = carry
        pltpu.sync_copy(edges_h.at[pl.ds(ch * 2 * CHUNK, 2 * CHUNK)], ebuf)

        def step(i, c):
            sv = ebuf[pl.ds(i * 16, 16)]
            dv = ebuf[pl.ds(CHUNK + i * 16, 16)]
            m = (dv & (NW - 1)) == g
            cs = plsc.cumsum(jnp.where(m, 1, 0))
            pos = c + cs - 1
            pw = pos & (CAP - 1)
            plsc.store_scatter(cdst, [pw >> BKLOG, pw & (BK - 1)],
                               dv >> 5, mask=m)
            plsc.store_scatter(csrc, [pw >> BKLOG, pw & (BK - 1)],
                               sv, mask=m)
            return c + _scalar(cs, 15)

        cur = lax.fori_loop(0, CHUNK // 16, step, cur)
        for j in range(BK // 16):
            p = cur + j * 16 + iota16
            pw = p & (CAP - 1)
            plsc.store_scatter(cdst, [pw >> BKLOG, pw & (BK - 1)],
                               jnp.full((16,), TRASH, jnp.int32), mask=ftrue)
            plsc.store_scatter(csrc, [pw >> BKLOG, pw & (BK - 1)],
                               jnp.zeros((16,), jnp.int32), mask=ftrue)
        extra = jnp.where(ch == NCHUNK - 1, BK - 1, 0)
        nb_d = (cur - done + extra) >> BKLOG

        def drain(t, _):
            batch(((done >> BKLOG) + t) & (NB2 - 1), 0)
            return 0

        lax.fori_loop(0, nb_d, drain, 0)
        return (cur, done + (nb_d << BKLOG))

    cur, done = lax.fori_loop(0, NCHUNK, chunk_one,
                              (jnp.int32(0), jnp.int32(0)))

    def obatch(b, _):
        pltpu.sync_copy(acc.at[pl.ds(b * 64, 64)], sum_out.at[zidx.at[b]])
        return 0

    lax.fori_loop(0, NZB, obatch, 0)


def _build_zidx(zidx, g):
    iota16 = lax.iota(jnp.int32, 16)
    ftrue = iota16 >= 0

    def zi(i, _):
        p = i * 16 + iota16
        plsc.store_scatter(zidx, [p >> 6, p & 63], g + NW * p,
                           mask=ftrue)
        return 0

    lax.fori_loop(0, RPW // 16, zi, 0)


def _sc1_body(feat_h, edges_h, sum_out, deg_out,
              ebuf, csrc, cdst, rows, acc, cnt, zidx, sem):
    c = lax.axis_index("c")
    s = lax.axis_index("s")
    g = s * NCORE + c
    _build_zidx(zidx, g)

    _agg_pass(feat_h, edges_h, sum_out, g, ebuf,
              csrc, cdst, rows, acc, zidx, sem, cnt=cnt)

    z16 = jnp.zeros((16,), jnp.float32)

    def dbatch(b, _):
        def crow(i, _):
            l = b * BK + i
            rows[i, pl.ds(0, 16)] = cnt[l >> 3, pl.ds((l & 7) * 16, 16)]
            for j in range(1, DEGW // 16):
                rows[i, pl.ds(j * 16, 16)] = z16
            return 0

        lax.fori_loop(0, 64, crow, 0)
        pltpu.sync_copy(rows.at[pl.ds(0, 64)], deg_out.at[zidx.at[b]])
        return 0

    lax.fori_loop(0, NZB, dbatch, 0)


_sc_agg1 = functools.partial(
    pl.kernel,
    out_type=(jax.ShapeDtypeStruct((NPAD, D), jnp.float32),
              jax.ShapeDtypeStruct((NPAD, DEGW), jnp.float32)),
    mesh=_mesh,
    compiler_params=pltpu.CompilerParams(needs_layout_passes=False),
    scratch_types=[
        pltpu.VMEM((2 * CHUNK,), jnp.int32),
        pltpu.VMEM((NB2, BK), jnp.int32),
        pltpu.VMEM((NB2, BK), jnp.int32),
        pltpu.VMEM((BK, D), jnp.float32),
        pltpu.VMEM((ACC_ROWS, D), jnp.float32),
        pltpu.VMEM((ACC_ROWS // 8, 128), jnp.float32),
        pltpu.VMEM((NZB, 64), jnp.int32),
        pltpu.SemaphoreType.DMA,
    ],
)(_sc1_body)


def _sc2_body(h0_h, h1_h, edges_h, s2a_out, s2b_out,
              ebuf, csrc, cdst, rows, acc, zidx, sem):
    c = lax.axis_index("c")
    s = lax.axis_index("s")
    g = s * NCORE + c
    _build_zidx(zidx, g)
    for feat_h, sout in ((h0_h, s2a_out), (h1_h, s2b_out)):
        _agg_pass(feat_h, edges_h, sout, g, ebuf,
                  csrc, cdst, rows, acc, zidx, sem)


_sc_agg2 = functools.partial(
    pl.kernel,
    out_type=(jax.ShapeDtypeStruct((NPAD, D), jnp.float32),
              jax.ShapeDtypeStruct((NPAD, D), jnp.float32)),
    mesh=_mesh,
    compiler_params=pltpu.CompilerParams(needs_layout_passes=False),
    scratch_types=[
        pltpu.VMEM((2 * CHUNK,), jnp.int32),
        pltpu.VMEM((NB2, BK), jnp.int32),
        pltpu.VMEM((NB2, BK), jnp.int32),
        pltpu.VMEM((BK, D), jnp.float32),
        pltpu.VMEM((ACC_ROWS, D), jnp.float32),
        pltpu.VMEM((NZB, 64), jnp.int32),
        pltpu.SemaphoreType.DMA,
    ],
)(_sc2_body)


BM = 512


def _dot(a, b):
    return lax.dot_general(a, b, (((1,), (0,)), ((), ())),
                           precision=lax.Precision.HIGHEST,
                           preferred_element_type=jnp.float32)


def _sigmoid(x):
    return 1.0 / (1.0 + jnp.exp(-x))


def _tc1_body(x_ref, s_ref, d_ref, w1a_ref, w1b_ref, b1_ref,
              h0_ref, h1_ref):
    inv_deg = 1.0 / jnp.maximum(d_ref[:, 0:1], 1.0)
    mean = s_ref[...] * inv_deg
    acc = _dot(x_ref[...], w1a_ref[...]) + _dot(mean, w1b_ref[...])
    h = _sigmoid(acc + b1_ref[...])
    h0_ref[...] = h[:, :D]
    h1_ref[...] = h[:, D:]


def _tc1(featp, sum1, deg, w1a, w1b, b1r):
    return pl.pallas_call(
        _tc1_body,
        grid=(NPAD // BM,),
        in_specs=[
            pl.BlockSpec((BM, D), lambda i: (i, 0)),
            pl.BlockSpec((BM, D), lambda i: (i, 0)),
            pl.BlockSpec((BM, DEGW), lambda i: (i, 0)),
            pl.BlockSpec((D, H0), lambda i: (0, 0)),
            pl.BlockSpec((D, H0), lambda i: (0, 0)),
            pl.BlockSpec((1, H0), lambda i: (0, 0)),
        ],
        out_specs=[pl.BlockSpec((BM, D), lambda i: (i, 0)),
                   pl.BlockSpec((BM, D), lambda i: (i, 0))],
        out_shape=[jax.ShapeDtypeStruct((NPAD, D), jnp.float32),
                   jax.ShapeDtypeStruct((NPAD, D), jnp.float32)],
    )(featp, sum1, deg, w1a, w1b, b1r)


def _tc2_body(h0_ref, h1_ref, sa_ref, sb_ref, d_ref,
              w2a0_ref, w2a1_ref, w2b0_ref, w2b1_ref, b2_ref,
              wm1_ref, bm1_ref, wm2_ref, bm2_ref, o_ref):
    inv_deg = 1.0 / jnp.maximum(d_ref[:, 0:1], 1.0)
    h2 = _sigmoid(_dot(h0_ref[...], w2a0_ref[...])
                  + _dot(h1_ref[...], w2a1_ref[...])
                  + _dot(sa_ref[...] * inv_deg, w2b0_ref[...])
                  + _dot(sb_ref[...] * inv_deg, w2b1_ref[...])
                  + b2_ref[...])
    z = jnp.maximum(_dot(h2, wm1_ref[...]) + bm1_ref[...], 0.0)
    o_ref[...] = _dot(z, wm2_ref[...]) + bm2_ref[...]


def _tc2(h0, h1, s2a, s2b, deg, w2a0, w2a1, w2b0, w2b1, b2r,
         wm1, bm1r, wm2, bm2r):
    row = lambda i: (i, 0)
    fix = lambda i: (0, 0)
    return pl.pallas_call(
        _tc2_body,
        grid=(NPAD // BM,),
        in_specs=[
            pl.BlockSpec((BM, D), row),
            pl.BlockSpec((BM, D), row),
            pl.BlockSpec((BM, D), row),
            pl.BlockSpec((BM, D), row),
            pl.BlockSpec((BM, DEGW), row),
            pl.BlockSpec((D, H1), fix),
            pl.BlockSpec((D, H1), fix),
            pl.BlockSpec((D, H1), fix),
            pl.BlockSpec((D, H1), fix),
            pl.BlockSpec((1, H1), fix),
            pl.BlockSpec((H1, H2), fix),
            pl.BlockSpec((1, H2), fix),
            pl.BlockSpec((H2, OUT), fix),
            pl.BlockSpec((1, OUT), fix),
        ],
        out_specs=pl.BlockSpec((BM, OUT), row),
        out_shape=jax.ShapeDtypeStruct((NPAD, OUT), jnp.float32),
    )(h0, h1, s2a, s2b, deg, w2a0, w2a1, w2b0, w2b1, b2r,
      wm1, bm1r, wm2, bm2r)


def kernel(features, edge_index, W1, b1, W2, b2, Wm1, bm1, Wm2, bm2):
    src = edge_index[0]
    dst = edge_index[1]
    e2 = jnp.concatenate([src.reshape(NCHUNK, CHUNK),
                          dst.reshape(NCHUNK, CHUNK)], axis=1).reshape(-1)
    featp = jnp.zeros((NPAD, D), jnp.float32).at[:N].set(features)

    sum1, deg = _sc_agg1(features, e2)
    h0, h1 = _tc1(featp, sum1, deg, W1[:D], W1[D:], b1.reshape(1, H0))
    s2a, s2b = _sc_agg2(h0, h1, e2)
    out = _tc2(h0, h1, s2a, s2b, deg,
               W2[0:D], W2[D:2 * D], W2[2 * D:3 * D], W2[3 * D:4 * D],
               b2.reshape(1, H1), Wm1, bm1.reshape(1, H2),
               Wm2, bm2.reshape(1, OUT))
    return out[:N]


# speedup vs baseline: 1.0555x; 1.0389x over previous
import functools

import jax
import jax.numpy as jnp
from jax import lax
from jax.experimental import pallas as pl
from jax.experimental.pallas import tpu as pltpu
from jax.experimental.pallas import tpu_sc as plsc

N = 10000
E = 160000
D = 256
H0 = 512
H1 = 256
H2 = 512
OUT = 64

NPAD = 10240
NTILE = 16
NCORE = 2
NW = NTILE * NCORE
RPW = NPAD // NW
ACC_ROWS = RPW + 8
TRASH = RPW
CHUNK = 3200
NCHUNK = E // CHUNK
BK = 64
BKLOG = 6
CAP = 4096
NB2 = CAP // BK
NZB = RPW // 64
DEGW = 256

_mesh = plsc.VectorSubcoreMesh(
    core_axis_name="c", subcore_axis_name="s", num_cores=NCORE,
    num_subcores=NTILE)


def _scalar(vec, i):
    return jnp.squeeze(lax.slice(vec, (i,), (i + 1,)))


def _agg_pass(feat_h, edges_h, sum_out, g, ebuf,
              csrc, cdst, rows, acc, zidx, sem, cnt=None):
    iota16 = lax.iota(jnp.int32, 16)
    ftrue = iota16 >= 0
    one16 = jnp.where(iota16 == 0, 1.0, 0.0)
    z16 = jnp.zeros((16,), jnp.float32)
    W = rows.shape[1]

    def zacc(i, _):
        for j in range(W // 16):
            acc[i, pl.ds(j * 16, 16)] = z16
        return 0

    lax.fori_loop(0, ACC_ROWS, zacc, 0)
    if cnt is not None:
        def zcnt(i, _):
            for j in range(8):
                cnt[i, pl.ds(j * 16, 16)] = z16
            return 0

        lax.fori_loop(0, ACC_ROWS // 8, zcnt, 0)

    def batch(br, _):
        pltpu.async_copy(feat_h.at[csrc.at[br]], rows, sem).wait()

        for sub in range(BK // 16):
            dv = cdst[br, pl.ds(sub * 16, 16)]
            for i in range(16):
                d = _scalar(dv, i)
                r = sub * 16 + i
                for j in range(W // 16):
                    plsc.addupdate(acc.at[d, pl.ds(j * 16, 16)],
                                   rows[r, pl.ds(j * 16, 16)])
                if cnt is not None:
                    plsc.addupdate(
                        cnt.at[d >> 3, pl.ds((d & 7) * 16, 16)], one16)
        return 0

    def chunk_one(ch, carry):
        cur, done = carry
        pltpu.sync_copy(edges_h.at[pl.ds(ch * 2 * CHUNK, 2 * CHUNK)], ebuf)

        def step(i, c):
            sv = ebuf[pl.ds(i * 16, 16)]
            dv = ebuf[pl.ds(CHUNK + i * 16, 16)]
            m = (dv & (NW - 1)) == g
            cs = plsc.cumsum(jnp.where(m, 1, 0))
            pos = c + cs - 1
            pw = pos & (CAP - 1)
            plsc.store_scatter(cdst, [pw >> BKLOG, pw & (BK - 1)],
                               dv >> 5, mask=m)
            plsc.store_scatter(csrc, [pw >> BKLOG, pw & (BK - 1)],
                               sv, mask=m)
            return c + _scalar(cs, 15)

        cur = lax.fori_loop(0, CHUNK // 16, step, cur)
        for j in range(BK // 16):
            p = cur + j * 16 + iota16
            pw = p & (CAP - 1)
            plsc.store_scatter(cdst, [pw >> BKLOG, pw & (BK - 1)],
                               jnp.full((16,), TRASH, jnp.int32), mask=ftrue)
            plsc.store_scatter(csrc, [pw >> BKLOG, pw & (BK - 1)],
                               jnp.zeros((16,), jnp.int32), mask=ftrue)
        extra = jnp.where(ch == NCHUNK - 1, BK - 1, 0)
        nb_d = (cur - done + extra) >> BKLOG

        def drain(t, _):
            batch(((done >> BKLOG) + t) & (NB2 - 1), 0)
            return 0

        lax.fori_loop(0, nb_d, drain, 0)
        return (cur, done + (nb_d << BKLOG))

    cur, done = lax.fori_loop(0, NCHUNK, chunk_one,
                              (jnp.int32(0), jnp.int32(0)))

    def obatch(b, _):
        pltpu.sync_copy(acc.at[pl.ds(b * 64, 64)], sum_out.at[zidx.at[b]])
        return 0

    lax.fori_loop(0, NZB, obatch, 0)


def _build_zidx(zidx, g):
    iota16 = lax.iota(jnp.int32, 16)
    ftrue = iota16 >= 0

    def zi(i, _):
        p = i * 16 + iota16
        plsc.store_scatter(zidx, [p >> 6, p & 63], g + NW * p,
                           mask=ftrue)
        return 0

    lax.fori_loop(0, RPW // 16, zi, 0)


def _sc1_body(feat_h, edges_h, sum_out, deg_out,
              ebuf, csrc, cdst, rows, acc, cnt, zidx, sem):
    c = lax.axis_index("c")
    s = lax.axis_index("s")
    g = s * NCORE + c
    _build_zidx(zidx, g)

    _agg_pass(feat_h, edges_h, sum_out, g, ebuf,
              csrc, cdst, rows, acc, zidx, sem, cnt=cnt)

    z16 = jnp.zeros((16,), jnp.float32)

    def dbatch(b, _):
        def crow(i, _):
            l = b * BK + i
            rows[i, pl.ds(0, 16)] = cnt[l >> 3, pl.ds((l & 7) * 16, 16)]
            for j in range(1, DEGW // 16):
                rows[i, pl.ds(j * 16, 16)] = z16
            return 0

        lax.fori_loop(0, 64, crow, 0)
        pltpu.sync_copy(rows.at[pl.ds(0, 64)], deg_out.at[zidx.at[b]])
        return 0

    lax.fori_loop(0, NZB, dbatch, 0)


_sc_agg1 = functools.partial(
    pl.kernel,
    out_type=(jax.ShapeDtypeStruct((NPAD, D), jnp.float32),
              jax.ShapeDtypeStruct((NPAD, DEGW), jnp.float32)),
    mesh=_mesh,
    compiler_params=pltpu.CompilerParams(needs_layout_passes=False),
    scratch_types=[
        pltpu.VMEM((2 * CHUNK,), jnp.int32),
        pltpu.VMEM((NB2, BK), jnp.int32),
        pltpu.VMEM((NB2, BK), jnp.int32),
        pltpu.VMEM((BK, D), jnp.float32),
        pltpu.VMEM((ACC_ROWS, D), jnp.float32),
        pltpu.VMEM((ACC_ROWS // 8, 128), jnp.float32),
        pltpu.VMEM((NZB, 64), jnp.int32),
        pltpu.SemaphoreType.DMA,
    ],
)(_sc1_body)


def _sc2_body(h0_h, h1_h, edges_h, s2a_out, s2b_out,
              ebuf, csrc, cdst, rows, acc, zidx, sem):
    c = lax.axis_index("c")
    s = lax.axis_index("s")
    g = s * NCORE + c
    _build_zidx(zidx, g)
    for feat_h, sout in ((h0_h, s2a_out), (h1_h, s2b_out)):
        _agg_pass(feat_h, edges_h, sout, g, ebuf,
                  csrc, cdst, rows, acc, zidx, sem)


_sc_agg2 = functools.partial(
    pl.kernel,
    out_type=(jax.ShapeDtypeStruct((NPAD, D), jnp.float32),
              jax.ShapeDtypeStruct((NPAD, D), jnp.float32)),
    mesh=_mesh,
    compiler_params=pltpu.CompilerParams(needs_layout_passes=False),
    scratch_types=[
        pltpu.VMEM((2 * CHUNK,), jnp.int32),
        pltpu.VMEM((NB2, BK), jnp.int32),
        pltpu.VMEM((NB2, BK), jnp.int32),
        pltpu.VMEM((BK, D), jnp.float32),
        pltpu.VMEM((ACC_ROWS, D), jnp.float32),
        pltpu.VMEM((NZB, 64), jnp.int32),
        pltpu.SemaphoreType.DMA,
    ],
)(_sc2_body)


BM = 512


def _dot(a, b):
    return lax.dot_general(a, b, (((1,), (0,)), ((), ())),
                           precision=lax.Precision.HIGHEST,
                           preferred_element_type=jnp.float32)


def _sigmoid(x):
    return 1.0 / (1.0 + jnp.exp(-x))


def _tc1_body(x_ref, s_ref, d_ref, w1a_ref, w1b_ref, b1_ref,
              h0_ref, h1_ref):
    inv_deg = 1.0 / jnp.maximum(d_ref[:, 0:1], 1.0)
    mean = s_ref[...] * inv_deg
    acc = _dot(x_ref[...], w1a_ref[...]) + _dot(mean, w1b_ref[...])
    h = _sigmoid(acc + b1_ref[...])
    h0_ref[...] = h[:, :D]
    h1_ref[...] = h[:, D:]


def _tc1(featp, sum1, deg, w1a, w1b, b1r):
    return pl.pallas_call(
        _tc1_body,
        grid=(NPAD // BM,),
        in_specs=[
            pl.BlockSpec((BM, D), lambda i: (i, 0)),
            pl.BlockSpec((BM, D), lambda i: (i, 0)),
            pl.BlockSpec((BM, DEGW), lambda i: (i, 0)),
            pl.BlockSpec((D, H0), lambda i: (0, 0)),
            pl.BlockSpec((D, H0), lambda i: (0, 0)),
            pl.BlockSpec((1, H0), lambda i: (0, 0)),
        ],
        out_specs=[pl.BlockSpec((BM, D), lambda i: (i, 0)),
                   pl.BlockSpec((BM, D), lambda i: (i, 0))],
        out_shape=[jax.ShapeDtypeStruct((NPAD, D), jnp.float32),
                   jax.ShapeDtypeStruct((NPAD, D), jnp.float32)],
    )(featp, sum1, deg, w1a, w1b, b1r)


def _tc2_body(h0_ref, h1_ref, sa_ref, sb_ref, d_ref,
              w2a0_ref, w2a1_ref, w2b0_ref, w2b1_ref, b2_ref,
              wm1_ref, bm1_ref, wm2_ref, bm2_ref, o_ref):
    inv_deg = 1.0 / jnp.maximum(d_ref[:, 0:1], 1.0)
    h2 = _sigmoid(_dot(h0_ref[...], w2a0_ref[...])
                  + _dot(h1_ref[...], w2a1_ref[...])
                  + _dot(sa_ref[...] * inv_deg, w2b0_ref[...])
                  + _dot(sb_ref[...] * inv_deg, w2b1_ref[...])
                  + b2_ref[...])
    z = jnp.maximum(_dot(h2, wm1_ref[...]) + bm1_ref[...], 0.0)
    o_ref[...] = _dot(z, wm2_ref[...]) + bm2_ref[...]


def _tc2(h0, h1, s2a, s2b, deg, w2a0, w2a1, w2b0, w2b1, b2r,
         wm1, bm1r, wm2, bm2r):
    row = lambda i: (i, 0)
    fix = lambda i: (0, 0)
    return pl.pallas_call(
        _tc2_body,
        grid=(NPAD // BM,),
        in_specs=[
            pl.BlockSpec((BM, D), row),
            pl.BlockSpec((BM, D), row),
            pl.BlockSpec((BM, D), row),
            pl.BlockSpec((BM, D), row),
            pl.BlockSpec((BM, DEGW), row),
            pl.BlockSpec((D, H1), fix),
            pl.BlockSpec((D, H1), fix),
            pl.BlockSpec((D, H1), fix),
            pl.BlockSpec((D, H1), fix),
            pl.BlockSpec((1, H1), fix),
            pl.BlockSpec((H1, H2), fix),
            pl.BlockSpec((1, H2), fix),
            pl.BlockSpec((H2, OUT), fix),
            pl.BlockSpec((1, OUT), fix),
        ],
        out_specs=pl.BlockSpec((BM, OUT), row),
        out_shape=jax.ShapeDtypeStruct((NPAD, OUT), jnp.float32),
    )(h0, h1, s2a, s2b, deg, w2a0, w2a1, w2b0, w2b1, b2r,
      wm1, bm1r, wm2, bm2r)


def kernel(features, edge_index, W1, b1, W2, b2, Wm1, bm1, Wm2, bm2):
    src = edge_index[0]
    dst = edge_index[1]
    e2 = jnp.concatenate([src.reshape(NCHUNK, CHUNK),
                          dst.reshape(NCHUNK, CHUNK)], axis=1).reshape(-1)
    featp = jnp.zeros((NPAD, D), jnp.float32).at[:N].set(features)

    sum1, deg = _sc_agg1(features, e2)
    h0, h1 = _tc1(featp, sum1, deg, W1[:D], W1[D:], b1.reshape(1, H0))
    s2a, s2b = _sc_agg2(h0, h1, e2)
    out = _tc2(h0, h1, s2a, s2b, deg,
               W2[0:D], W2[D:2 * D], W2[2 * D:3 * D], W2[3 * D:4 * D],
               b2.reshape(1, H1), Wm1, bm1.reshape(1, H2),
               Wm2, bm2.reshape(1, OUT))
    return out[:N]
